# baseline (speedup 1.0000x reference)
import functools

import jax
import jax.numpy as jnp
from jax import lax
from jax.experimental import pallas as pl
from jax.experimental.pallas import tpu as pltpu
from jax.experimental.pallas import tpu_sc as plsc

N = 10000
E = 320000
G = 64
NP = 10240
EP = 327680
CH = 128
NC, NS = 2, 16
K1 = EP // (NC * NS * CH)
K2 = EP // (NS * CH)
RPT = NP // NS
D1 = 136
D2 = 256
DQ = 64
NQ = 4
BR = 1280
GRID = NP // BR

_mesh = plsc.VectorSubcoreMesh(
    core_axis_name="c", subcore_axis_name="s", num_cores=NC, num_subcores=NS)
_sc_params = pltpu.CompilerParams(use_tc_tiling_on_sc=False)


@functools.partial(
    pl.kernel,
    out_type=(jax.ShapeDtypeStruct((NC * NP, DQ), jnp.float32),
              jax.ShapeDtypeStruct((NP, 8), jnp.float32),
              jax.ShapeDtypeStruct((NP, 8), jnp.float32)),
    mesh=_mesh,
    scratch_types=[
        pltpu.VMEM((K2, CH), jnp.int32),
        pltpu.VMEM((K2, CH), jnp.int32),
        pltpu.VMEM((CH, DQ), jnp.float32),
        pltpu.VMEM((CH, DQ), jnp.float32),
        pltpu.VMEM((CH, 8), jnp.float32),
        pltpu.VMEM((CH, 8), jnp.float32),
        pltpu.VMEM_SHARED((NP, DQ), jnp.float32),
        pltpu.VMEM_SHARED((NC * NP, 8), jnp.float32),
        pltpu.SemaphoreType.DMA,
        pltpu.SemaphoreType.DMA,
        pltpu.SemaphoreType.DMA,
    ],
    compiler_params=_sc_params,
)
def _sc_pass1(featA, featB, gidx, sidx, zq, z8, ones8,
              out_a, out_b, out_deg,
              gslab, sslab, buf0, buf1, bufb, onesb, acc, acc8,
              sem0, sem1, semb):
    c = lax.axis_index("c")
    s = lax.axis_index("s")
    r0 = s * RPT
    pltpu.sync_copy(zq.at[pl.ds(r0, RPT)], acc.at[pl.ds(r0, RPT)])
    pltpu.sync_copy(z8.at[pl.ds(r0, RPT)], acc8.at[pl.ds(c * NP + r0, RPT)])
    pltpu.sync_copy(ones8, onesb)
    pltpu.sync_copy(gidx.at[c * NS + s], gslab)
    pltpu.sync_copy(sidx.at[s], sslab)
    plsc.subcore_barrier()

    pltpu.async_copy(featA.at[gslab.at[0]], buf0, sem0)

    def small(j):
        @pl.when(c == 0)
        def _():
            pltpu.async_copy(featB.at[gslab.at[j]], bufb, semb).wait()
            pltpu.sync_copy(bufb, acc8.at[sslab.at[j]], add=True)

        @pl.when(c == 1)
        def _():
            pltpu.sync_copy(onesb, acc8.at[gslab.at[j]], add=True)

    def body(j2, carry):
        j = 2 * j2
        pltpu.async_copy(featA.at[gslab.at[j + 1]], buf1, sem1)
        pltpu.make_async_copy(featA.at[gslab.at[j]], buf0, sem0).wait()
        pltpu.sync_copy(buf0, acc.at[sslab.at[j]], add=True)
        small(j)

        @pl.when(j + 2 < K2)
        def _():
            pltpu.async_copy(featA.at[gslab.at[j + 2]], buf0, sem0)

        pltpu.make_async_copy(featA.at[gslab.at[j + 1]], buf1, sem1).wait()
        pltpu.sync_copy(buf1, acc.at[sslab.at[j + 1]], add=True)
        small(j + 1)
        return carry

    lax.fori_loop(0, K2 // 2, body, 0)
    plsc.subcore_barrier()
    pltpu.sync_copy(acc.at[pl.ds(r0, RPT)], out_a.at[pl.ds(c * NP + r0, RPT)])

    @pl.when(c == 0)
    def _():
        pltpu.sync_copy(acc8.at[pl.ds(r0, RPT)], out_b.at[pl.ds(r0, RPT)])

    @pl.when(c == 1)
    def _():
        pltpu.sync_copy(acc8.at[pl.ds(NP + r0, RPT)],
                        out_deg.at[pl.ds(r0, RPT)])


@functools.partial(
    pl.kernel,
    out_type=jax.ShapeDtypeStruct((NQ * NP, DQ), jnp.float32),
    mesh=_mesh,
    scratch_types=[
        pltpu.VMEM((K2, CH), jnp.int32),
        pltpu.VMEM((K2, CH), jnp.int32),
        pltpu.VMEM((CH, DQ), jnp.float32),
        pltpu.VMEM((CH, DQ), jnp.float32),
        pltpu.VMEM_SHARED((NP, DQ), jnp.float32),
        pltpu.SemaphoreType.DMA,
        pltpu.SemaphoreType.DMA,
    ],
    compiler_params=_sc_params,
)
def _sc_agg(table, gidx, sidx, zq, out, gslab, sslab, buf0, buf1, acc,
            sem0, sem1):
    c = lax.axis_index("c")
    s = lax.axis_index("s")
    r0 = s * RPT
    pltpu.sync_copy(sidx.at[s], sslab)
    for q in range(2):
        v = 2 * q + c
        pltpu.sync_copy(gidx.at[v * NS + s], gslab)
        pltpu.sync_copy(zq.at[pl.ds(r0, RPT)], acc.at[pl.ds(r0, RPT)])
        plsc.subcore_barrier()

        pltpu.async_copy(table.at[gslab.at[0]], buf0, sem0)

        def body(j2, carry):
            j = 2 * j2
            pltpu.async_copy(table.at[gslab.at[j + 1]], buf1, sem1)
            pltpu.make_async_copy(table.at[gslab.at[j]], buf0, sem0).wait()
            pltpu.sync_copy(buf0, acc.at[sslab.at[j]], add=True)

            @pl.when(j + 2 < K2)
            def _():
                pltpu.async_copy(table.at[gslab.at[j + 2]], buf0, sem0)

            pltpu.make_async_copy(table.at[gslab.at[j + 1]], buf1,
                                  sem1).wait()
            pltpu.sync_copy(buf1, acc.at[sslab.at[j + 1]], add=True)
            return carry

        lax.fori_loop(0, K2 // 2, body, 0)
        plsc.subcore_barrier()
        pltpu.sync_copy(acc.at[pl.ds(r0, RPT)],
                        out.at[pl.ds(v * NP + r0, RPT)])


def _split_q(ms_ref, ms):
    for j in range(NQ):
        ms_ref[j, :, :] = ms[:, j * DQ:(j + 1) * DQ]


def _cat_q(ref):
    return jnp.concatenate([ref[j] for j in range(NQ)], axis=1)


def _HI():
    return dict(preferred_element_type=jnp.float32,
                precision=lax.Precision.HIGHEST)


def _tcn_body(sa_ref, sb_ref, degs_ref, hsa_ref, hsb_ref, rdeg_ref):
    sa = jnp.concatenate([sa_ref[0], sa_ref[1]], axis=1)
    sb = sb_ref[...]
    s2 = (jnp.sum(sa * sa, axis=1, keepdims=True)
          + jnp.sum(sb * sb, axis=1, keepdims=True))
    inv = lax.rsqrt(jnp.where(s2 > 0, s2, 1.0))
    deg = degs_ref[:, 0:1] + 1.0
    rdeg = lax.rsqrt(deg)
    sc = inv * rdeg
    hs128 = sa * sc
    hsa_ref[0, :, :] = hs128[:, :DQ]
    hsa_ref[1, :, :] = hs128[:, DQ:]
    hsb_ref[...] = sb * sc
    rdeg_ref[...] = jnp.broadcast_to(rdeg, (BR, 128))


def _tc_norm(sumsa, sumsb, degs):
    return pl.pallas_call(
        _tcn_body,
        grid=(GRID,),
        in_specs=[
            pl.BlockSpec((NC, BR, DQ), lambda i: (0, i, 0)),
            pl.BlockSpec((BR, 8), lambda i: (i, 0)),
            pl.BlockSpec((BR, 8), lambda i: (i, 0)),
        ],
        out_specs=[
            pl.BlockSpec((NC, BR, DQ), lambda i: (0, i, 0)),
            pl.BlockSpec((BR, 8), lambda i: (i, 0)),
            pl.BlockSpec((BR, 128), lambda i: (i, 0)),
        ],
        out_shape=[
            jax.ShapeDtypeStruct((NC, NP, DQ), jnp.float32),
            jax.ShapeDtypeStruct((NP, 8), jnp.float32),
            jax.ShapeDtypeStruct((NP, 128), jnp.float32),
        ],
    )(sumsa, sumsb, degs)


def _tcl1_body(agga_ref, aggb_ref, hsa_ref, hsb_ref, rdeg_ref, w_ref, b_ref,
               hs_ref):
    rdeg = rdeg_ref[:, 0:1]
    pre = jnp.concatenate(
        [agga_ref[0] + hsa_ref[0], agga_ref[1] + hsa_ref[1],
         aggb_ref[...] + hsb_ref[...]], axis=1)
    m = jnp.dot(pre, w_ref[...], **_HI())
    h = jnp.maximum(m * rdeg + b_ref[0:1, :], 0.0)
    _split_q(hs_ref, h * rdeg)


def _tc_layer1(agga, aggb, hsa, hsb, rdegb, w0p, b0):
    return pl.pallas_call(
        _tcl1_body,
        grid=(GRID,),
        in_specs=[
            pl.BlockSpec((NC, BR, DQ), lambda i: (0, i, 0)),
            pl.BlockSpec((BR, 8), lambda i: (i, 0)),
            pl.BlockSpec((NC, BR, DQ), lambda i: (0, i, 0)),
            pl.BlockSpec((BR, 8), lambda i: (i, 0)),
            pl.BlockSpec((BR, 128), lambda i: (i, 0)),
            pl.BlockSpec((D1, D2), lambda i: (0, 0)),
            pl.BlockSpec((1, D2), lambda i: (0, 0)),
        ],
        out_specs=pl.BlockSpec((NQ, BR, DQ), lambda i: (0, i, 0)),
        out_shape=jax.ShapeDtypeStruct((NQ, NP, DQ), jnp.float32),
    )(agga, aggb, hsa, hsb, rdegb, w0p, b0)


def _tck_body(agg_ref, hs_ref, rdeg_ref, w_ref, b_ref, out_ref):
    rdeg = rdeg_ref[:, 0:1]
    pre = _cat_q(agg_ref) + _cat_q(hs_ref)
    m = jnp.dot(pre, w_ref[...], **_HI())
    h = jnp.maximum(m * rdeg + b_ref[0:1, :], 0.0)
    _split_q(out_ref, h * rdeg)


def _tc_layer(agg, hs, rdegb, w, b):
    return pl.pallas_call(
        _tck_body,
        grid=(GRID,),
        in_specs=[
            pl.BlockSpec((NQ, BR, DQ), lambda i: (0, i, 0)),
            pl.BlockSpec((NQ, BR, DQ), lambda i: (0, i, 0)),
            pl.BlockSpec((BR, 128), lambda i: (i, 0)),
            pl.BlockSpec((D2, D2), lambda i: (0, 0)),
            pl.BlockSpec((1, D2), lambda i: (0, 0)),
        ],
        out_specs=pl.BlockSpec((NQ, BR, DQ), lambda i: (0, i, 0)),
        out_shape=jax.ShapeDtypeStruct((NQ, NP, DQ), jnp.float32),
    )(agg, hs, rdegb, w, b)


def _tcf_body(agg_ref, hs_ref, rdeg_ref, w_ref, b_ref, wl_ref, bl_ref,
              batch_ref, out_ref, pacc, cacc):
    i = pl.program_id(0)
    rdeg = rdeg_ref[:, 0:1]
    pre = _cat_q(agg_ref) + _cat_q(hs_ref)
    m = jnp.dot(pre, w_ref[...], **_HI())
    h = jnp.maximum(m * rdeg + b_ref[0:1, :], 0.0)
    r = jnp.dot(h, wl_ref[...], **_HI())
    bt = batch_ref[0, 0, :]
    gid = lax.broadcasted_iota(jnp.int32, (BR, G), 1)
    oh = (bt[:, None] == gid).astype(jnp.float32)
    pc = lax.dot_general(oh, r, (((0,), (0,)), ((), ())), **_HI())
    ones = jnp.ones((BR, 128), jnp.float32)
    cn = lax.dot_general(oh, ones, (((0,), (0,)), ((), ())), **_HI())

    @pl.when(i == 0)
    def _():
        pacc[...] = jnp.zeros((G, 128), jnp.float32)
        cacc[...] = jnp.zeros((G, 128), jnp.float32)

    pacc[...] += pc
    cacc[...] += cn

    @pl.when(i == GRID - 1)
    def _():
        out_ref[...] = pacc[...] / jnp.maximum(cacc[...], 1.0) + bl_ref[...]


def _tc_final(agg, hs, rdegb, w, b, wlp, blb, batchp):
    return pl.pallas_call(
        _tcf_body,
        grid=(GRID,),
        in_specs=[
            pl.BlockSpec((NQ, BR, DQ), lambda i: (0, i, 0)),
            pl.BlockSpec((NQ, BR, DQ), lambda i: (0, i, 0)),
            pl.BlockSpec((BR, 128), lambda i: (i, 0)),
            pl.BlockSpec((D2, D2), lambda i: (0, 0)),
            pl.BlockSpec((1, D2), lambda i: (0, 0)),
            pl.BlockSpec((D2, 128), lambda i: (0, 0)),
            pl.BlockSpec((1, 128), lambda i: (0, 0)),
            pl.BlockSpec((1, 1, BR), lambda i: (i, 0, 0)),
        ],
        out_specs=pl.BlockSpec((G, 128), lambda i: (0, 0)),
        out_shape=jax.ShapeDtypeStruct((G, 128), jnp.float32),
        scratch_shapes=[
            pltpu.VMEM((G, 128), jnp.float32),
            pltpu.VMEM((G, 128), jnp.float32),
        ],
    )(agg, hs, rdegb, w, b, wlp, blb, batchp)


def kernel(x, pos, z, W0, b0, W1, b1, W2, b2, Wl, bl, edge_index, batch):
    f32 = jnp.float32
    xh = jnp.pad(x, ((0, NP - N), (0, 0))).reshape(
        NP, NC, DQ).transpose(1, 0, 2).reshape(NC * NP, DQ)
    featsB = jnp.pad(jnp.concatenate([pos, z[:, None]], axis=1),
                     ((0, NP - N), (0, 4)))
    src = jnp.pad(edge_index[0], (0, EP - E), constant_values=N)
    dst = jnp.pad(edge_index[1], (0, EP - E), constant_values=N)
    offs = (jnp.arange(NQ, dtype=jnp.int32) * NP)[:, None]
    gidx1 = (dst[None, :] + offs[:NC]).reshape(NC * NS, K2, CH)
    sidx1 = src.reshape(NS, K2, CH)
    gidx2h = (src[None, :] + offs[:NC]).reshape(NC * NS, K2, CH)
    gidx2 = (src[None, :] + offs).reshape(NQ * NS, K2, CH)
    sidx2 = dst.reshape(NS, K2, CH)
    z8 = jnp.zeros((NP, 8), f32)
    zq = jnp.zeros((NP, DQ), f32)
    ones8 = jnp.zeros((CH, 8), f32).at[:, 0].set(1.0)
    W0p = jnp.pad(W0, ((0, D1 - 132), (0, 0)))
    Wlp = jnp.pad(Wl, ((0, 0), (0, 127)))
    blb = jnp.broadcast_to(bl.reshape(1, 1), (1, 128))
    batchP = jnp.pad(batch, (0, NP - N), constant_values=127).reshape(
        GRID, 1, BR)

    out_a, out_b, out_deg = _sc_pass1(xh, featsB, gidx1, sidx1,
                                      zq, z8, ones8)
    hsa, hsb, rdegb = _tc_norm(out_a.reshape(NC, NP, DQ), out_b, out_deg)

    agga, aggb, _ = _sc_pass1(hsa.reshape(NC * NP, DQ), hsb, gidx2h, sidx2,
                              zq, z8, ones8)
    hs1 = _tc_layer1(agga.reshape(NC, NP, DQ), aggb, hsa, hsb, rdegb,
                     W0p, b0.reshape(1, D2))

    agg1 = _sc_agg(hs1.reshape(NQ * NP, DQ), gidx2, sidx2, zq)
    hs2 = _tc_layer(agg1.reshape(NQ, NP, DQ), hs1, rdegb, W1,
                    b1.reshape(1, D2))
    agg2 = _sc_agg(hs2.reshape(NQ * NP, DQ), gidx2, sidx2, zq)
    res = _tc_final(agg2.reshape(NQ, NP, DQ), hs2, rdegb, W2,
                    b2.reshape(1, D2), Wlp, blb, batchP)
    return res[:, 0]


# speedup vs baseline: 1.0082x; 1.0082x over previous
import functools

import jax
import jax.numpy as jnp
from jax import lax
from jax.experimental import pallas as pl
from jax.experimental.pallas import tpu as pltpu
from jax.experimental.pallas import tpu_sc as plsc

N = 10000
E = 320000
G = 64
NP = 10240
EP = 327680
CH = 128
NC, NS = 2, 16
K1 = EP // (NC * NS * CH)
K2 = EP // (NS * CH)
RPT = NP // NS
D1 = 136
D2 = 256
DQ = 64
NQ = 4
BR = 1280
GRID = NP // BR

_mesh = plsc.VectorSubcoreMesh(
    core_axis_name="c", subcore_axis_name="s", num_cores=NC, num_subcores=NS)
_sc_params = pltpu.CompilerParams(use_tc_tiling_on_sc=False)


@functools.partial(
    pl.kernel,
    out_type=(jax.ShapeDtypeStruct((NC * NP, DQ), jnp.float32),
              jax.ShapeDtypeStruct((NP, 8), jnp.float32),
              jax.ShapeDtypeStruct((NP, 8), jnp.float32)),
    mesh=_mesh,
    scratch_types=[
        pltpu.VMEM((K2, CH), jnp.int32),
        pltpu.VMEM((K2, CH), jnp.int32),
        pltpu.VMEM((CH, DQ), jnp.float32),
        pltpu.VMEM((CH, DQ), jnp.float32),
        pltpu.VMEM((CH, 8), jnp.float32),
        pltpu.VMEM((CH, 8), jnp.float32),
        pltpu.VMEM_SHARED((NP, DQ), jnp.float32),
        pltpu.VMEM_SHARED((NC * NP, 8), jnp.float32),
        pltpu.SemaphoreType.DMA,
        pltpu.SemaphoreType.DMA,
        pltpu.SemaphoreType.DMA,
    ],
    compiler_params=_sc_params,
)
def _sc_pass1(featA, featB, gidx, sidx, zq, z8, ones8,
              out_a, out_b, out_deg,
              gslab, sslab, buf0, buf1, bufb, onesb, acc, acc8,
              sem0, sem1, semb):
    c = lax.axis_index("c")
    s = lax.axis_index("s")
    r0 = s * RPT
    pltpu.sync_copy(zq.at[pl.ds(r0, RPT)], acc.at[pl.ds(r0, RPT)])
    pltpu.sync_copy(z8.at[pl.ds(r0, RPT)], acc8.at[pl.ds(c * NP + r0, RPT)])
    pltpu.sync_copy(ones8, onesb)
    pltpu.sync_copy(gidx.at[c * NS + s], gslab)
    pltpu.sync_copy(sidx.at[s], sslab)
    plsc.subcore_barrier()

    pltpu.async_copy(featA.at[gslab.at[0]], buf0, sem0)

    def small(j):
        @pl.when(c == 0)
        def _():
            pltpu.async_copy(featB.at[gslab.at[j]], bufb, semb).wait()
            pltpu.sync_copy(bufb, acc8.at[sslab.at[j]], add=True)

        @pl.when(c == 1)
        def _():
            pltpu.sync_copy(onesb, acc8.at[gslab.at[j]], add=True)

    def body(j2, carry):
        j = 2 * j2
        pltpu.async_copy(featA.at[gslab.at[j + 1]], buf1, sem1)
        pltpu.make_async_copy(featA.at[gslab.at[j]], buf0, sem0).wait()
        pltpu.sync_copy(buf0, acc.at[sslab.at[j]], add=True)
        small(j)

        @pl.when(j + 2 < K2)
        def _():
            pltpu.async_copy(featA.at[gslab.at[j + 2]], buf0, sem0)

        pltpu.make_async_copy(featA.at[gslab.at[j + 1]], buf1, sem1).wait()
        pltpu.sync_copy(buf1, acc.at[sslab.at[j + 1]], add=True)
        small(j + 1)
        return carry

    lax.fori_loop(0, K2 // 2, body, 0)
    plsc.subcore_barrier()
    pltpu.sync_copy(acc.at[pl.ds(r0, RPT)], out_a.at[pl.ds(c * NP + r0, RPT)])

    @pl.when(c == 0)
    def _():
        pltpu.sync_copy(acc8.at[pl.ds(r0, RPT)], out_b.at[pl.ds(r0, RPT)])

    @pl.when(c == 1)
    def _():
        pltpu.sync_copy(acc8.at[pl.ds(NP + r0, RPT)],
                        out_deg.at[pl.ds(r0, RPT)])


@functools.partial(
    pl.kernel,
    out_type=jax.ShapeDtypeStruct((NQ * NP, DQ), jnp.float32),
    mesh=_mesh,
    scratch_types=[
        pltpu.VMEM((K2, CH), jnp.int32),
        pltpu.VMEM((K2, CH), jnp.int32),
        pltpu.VMEM((CH, DQ), jnp.float32),
        pltpu.VMEM((CH, DQ), jnp.float32),
        pltpu.VMEM_SHARED((NP, DQ), jnp.float32),
        pltpu.SemaphoreType.DMA,
        pltpu.SemaphoreType.DMA,
    ],
    compiler_params=_sc_params,
)
def _sc_agg(table, gidx, sidx, zq, out, gslab, sslab, buf0, buf1, acc,
            sem0, sem1):
    c = lax.axis_index("c")
    s = lax.axis_index("s")
    r0 = s * RPT
    pltpu.sync_copy(sidx.at[s], sslab)
    for q in range(2):
        v = 2 * q + c
        pltpu.sync_copy(gidx.at[v * NS + s], gslab)
        pltpu.sync_copy(zq.at[pl.ds(r0, RPT)], acc.at[pl.ds(r0, RPT)])
        plsc.subcore_barrier()

        pltpu.async_copy(table.at[gslab.at[0]], buf0, sem0)

        def body(j2, carry):
            j = 2 * j2
            pltpu.async_copy(table.at[gslab.at[j + 1]], buf1, sem1)
            pltpu.make_async_copy(table.at[gslab.at[j]], buf0, sem0).wait()
            pltpu.sync_copy(buf0, acc.at[sslab.at[j]], add=True)

            @pl.when(j + 2 < K2)
            def _():
                pltpu.async_copy(table.at[gslab.at[j + 2]], buf0, sem0)

            pltpu.make_async_copy(table.at[gslab.at[j + 1]], buf1,
                                  sem1).wait()
            pltpu.sync_copy(buf1, acc.at[sslab.at[j + 1]], add=True)
            return carry

        lax.fori_loop(0, K2 // 2, body, 0)
        plsc.subcore_barrier()
        pltpu.sync_copy(acc.at[pl.ds(r0, RPT)],
                        out.at[pl.ds(v * NP + r0, RPT)])


def _split_q(ms_ref, ms):
    for j in range(NQ):
        ms_ref[j, :, :] = ms[:, j * DQ:(j + 1) * DQ]


def _cat_q(ref):
    return jnp.concatenate([ref[j] for j in range(NQ)], axis=1)


def _tc0_body(sa_ref, sb_ref, degs_ref, w_ref, ms_ref, rdeg_ref):
    sa = jnp.concatenate([sa_ref[0], sa_ref[1]], axis=1)
    sb = sb_ref[...]
    s2 = (jnp.sum(sa * sa, axis=1, keepdims=True)
          + jnp.sum(sb * sb, axis=1, keepdims=True))
    h = jnp.concatenate([sa, sb], axis=1) * lax.rsqrt(
        jnp.where(s2 > 0, s2, 1.0))
    deg = degs_ref[:, 0:1] + 1.0
    rdeg = lax.rsqrt(deg)
    m = jnp.dot(h, w_ref[...], preferred_element_type=jnp.float32)
    _split_q(ms_ref, m * rdeg)
    rdeg_ref[...] = jnp.broadcast_to(rdeg, (BR, 128))


def _tc_layer0(sumsa, sumsb, degs, w0p):
    return pl.pallas_call(
        _tc0_body,
        grid=(GRID,),
        in_specs=[
            pl.BlockSpec((NC, BR, DQ), lambda i: (0, i, 0)),
            pl.BlockSpec((BR, 8), lambda i: (i, 0)),
            pl.BlockSpec((BR, 8), lambda i: (i, 0)),
            pl.BlockSpec((D1, D2), lambda i: (0, 0)),
        ],
        out_specs=[
            pl.BlockSpec((NQ, BR, DQ), lambda i: (0, i, 0)),
            pl.BlockSpec((BR, 128), lambda i: (i, 0)),
        ],
        out_shape=[
            jax.ShapeDtypeStruct((NQ, NP, DQ), jnp.float32),
            jax.ShapeDtypeStruct((NP, 128), jnp.float32),
        ],
    )(sumsa, sumsb, degs, w0p)


def _tck_body(agg_ref, msp_ref, rdeg_ref, w_ref, b_ref, ms_ref):
    rdeg = rdeg_ref[:, 0:1]
    pre = (_cat_q(agg_ref) + _cat_q(msp_ref)) * rdeg + b_ref[0:1, :]
    h = jnp.maximum(pre, 0.0)
    m = jnp.dot(h, w_ref[...], preferred_element_type=jnp.float32)
    _split_q(ms_ref, m * rdeg)


def _tc_layer(agg, msp, rdegb, w, b):
    return pl.pallas_call(
        _tck_body,
        grid=(GRID,),
        in_specs=[
            pl.BlockSpec((NQ, BR, DQ), lambda i: (0, i, 0)),
            pl.BlockSpec((NQ, BR, DQ), lambda i: (0, i, 0)),
            pl.BlockSpec((BR, 128), lambda i: (i, 0)),
            pl.BlockSpec((D2, D2), lambda i: (0, 0)),
            pl.BlockSpec((1, D2), lambda i: (0, 0)),
        ],
        out_specs=pl.BlockSpec((NQ, BR, DQ), lambda i: (0, i, 0)),
        out_shape=jax.ShapeDtypeStruct((NQ, NP, DQ), jnp.float32),
    )(agg, msp, rdegb, w, b)


def _tcf_body(agg_ref, msp_ref, rdeg_ref, b_ref, wl_ref, bl_ref, batch_ref,
              out_ref, pacc, cacc):
    i = pl.program_id(0)
    rdeg = rdeg_ref[:, 0:1]
    pre = (_cat_q(agg_ref) + _cat_q(msp_ref)) * rdeg + b_ref[0:1, :]
    h = jnp.maximum(pre, 0.0)
    bt = batch_ref[0, 0, :]
    gid = lax.broadcasted_iota(jnp.int32, (BR, G), 1)
    oh = (bt[:, None] == gid).astype(jnp.float32)
    pc = lax.dot_general(oh, h, (((0,), (0,)), ((), ())),
                         preferred_element_type=jnp.float32,
                         precision=lax.Precision.HIGHEST)
    ones = jnp.ones((BR, 128), jnp.float32)
    cn = lax.dot_general(oh, ones, (((0,), (0,)), ((), ())),
                         preferred_element_type=jnp.float32,
                         precision=lax.Precision.HIGHEST)

    @pl.when(i == 0)
    def _():
        pacc[...] = jnp.zeros((G, D2), jnp.float32)
        cacc[...] = jnp.zeros((G, 128), jnp.float32)

    pacc[...] += pc
    cacc[...] += cn

    @pl.when(i == GRID - 1)
    def _():
        pooled = pacc[...] / jnp.maximum(cacc[:, 0:1], 1.0)
        out_ref[...] = jnp.dot(
            pooled, wl_ref[...],
            preferred_element_type=jnp.float32) + bl_ref[...]


def _tc_final(agg, msp, rdegb, b, wlp, blb, batchp):
    return pl.pallas_call(
        _tcf_body,
        grid=(GRID,),
        in_specs=[
            pl.BlockSpec((NQ, BR, DQ), lambda i: (0, i, 0)),
            pl.BlockSpec((NQ, BR, DQ), lambda i: (0, i, 0)),
            pl.BlockSpec((BR, 128), lambda i: (i, 0)),
            pl.BlockSpec((1, D2), lambda i: (0, 0)),
            pl.BlockSpec((D2, 128), lambda i: (0, 0)),
            pl.BlockSpec((1, 128), lambda i: (0, 0)),
            pl.BlockSpec((1, 1, BR), lambda i: (i, 0, 0)),
        ],
        out_specs=pl.BlockSpec((G, 128), lambda i: (0, 0)),
        out_shape=jax.ShapeDtypeStruct((G, 128), jnp.float32),
        scratch_shapes=[
            pltpu.VMEM((G, D2), jnp.float32),
            pltpu.VMEM((G, 128), jnp.float32),
        ],
    )(agg, msp, rdegb, b, wlp, blb, batchp)


def kernel(x, pos, z, W0, b0, W1, b1, W2, b2, Wl, bl, edge_index, batch):
    f32 = jnp.float32
    xh = jnp.pad(x, ((0, NP - N), (0, 0))).reshape(
        NP, NC, DQ).transpose(1, 0, 2).reshape(NC * NP, DQ)
    featsB = jnp.pad(jnp.concatenate([pos, z[:, None]], axis=1),
                     ((0, NP - N), (0, 4)))
    src = jnp.pad(edge_index[0], (0, EP - E), constant_values=N)
    dst = jnp.pad(edge_index[1], (0, EP - E), constant_values=N)
    gidx1 = (dst[None, :]
             + (jnp.arange(NC, dtype=jnp.int32) * NP)[:, None]).reshape(
                 NC * NS, K2, CH)
    sidx1 = src.reshape(NS, K2, CH)
    gidx2 = (src[None, :]
             + (jnp.arange(NQ, dtype=jnp.int32) * NP)[:, None]).reshape(
                 NQ * NS, K2, CH)
    sidx2 = dst.reshape(NS, K2, CH)
    z8 = jnp.zeros((NP, 8), f32)
    zq = jnp.zeros((NP, DQ), f32)
    ones8 = jnp.zeros((CH, 8), f32).at[:, 0].set(1.0)
    W0p = jnp.pad(W0, ((0, D1 - 132), (0, 0)))
    Wlp = jnp.pad(Wl, ((0, 0), (0, 127)))
    blb = jnp.broadcast_to(bl.reshape(1, 1), (1, 128))
    batchP = jnp.pad(batch, (0, NP - N), constant_values=127).reshape(
        GRID, 1, BR)

    out_a, out_b, out_deg = _sc_pass1(xh, featsB, gidx1, sidx1,
                                      zq, z8, ones8)
    ms, rdegb = _tc_layer0(out_a.reshape(NC, NP, DQ), out_b, out_deg, W0p)

    for W_, b_ in ((W1, b0), (W2, b1)):
        agg = _sc_agg(ms.reshape(NQ * NP, DQ), gidx2, sidx2, zq)
        ms = _tc_layer(agg.reshape(NQ, NP, DQ), ms, rdegb, W_,
                       b_.reshape(1, D2))

    agg = _sc_agg(ms.reshape(NQ * NP, DQ), gidx2, sidx2, zq)
    res = _tc_final(agg.reshape(NQ, NP, DQ), ms, rdegb, b2.reshape(1, D2),
                    Wlp, blb, batchP)
    return res[:, 0]


# speedup vs baseline: 1.0141x; 1.0059x over previous
import functools

import jax
import jax.numpy as jnp
from jax import lax
from jax.experimental import pallas as pl
from jax.experimental.pallas import tpu as pltpu
from jax.experimental.pallas import tpu_sc as plsc

N = 10000
E = 320000
G = 64
NP = 10240
EP = 327680
CH = 128
NC, NS = 2, 16
K1 = EP // (NC * NS * CH)
K2 = EP // (NS * CH)
RPT = NP // NS
D1 = 136
D2 = 256
DQ = 64
NQ = 4
BR = 1280
GRID = NP // BR

_mesh = plsc.VectorSubcoreMesh(
    core_axis_name="c", subcore_axis_name="s", num_cores=NC, num_subcores=NS)
_sc_params = pltpu.CompilerParams(use_tc_tiling_on_sc=False)


@functools.partial(
    pl.kernel,
    out_type=(jax.ShapeDtypeStruct((NC * NP, DQ), jnp.float32),
              jax.ShapeDtypeStruct((NP, 8), jnp.float32),
              jax.ShapeDtypeStruct((NP, 8), jnp.float32)),
    mesh=_mesh,
    scratch_types=[
        pltpu.VMEM((K2, CH), jnp.int32),
        pltpu.VMEM((K2, CH), jnp.int32),
        pltpu.VMEM((CH, DQ), jnp.float32),
        pltpu.VMEM((CH, DQ), jnp.float32),
        pltpu.VMEM((CH, 8), jnp.float32),
        pltpu.VMEM((CH, 8), jnp.float32),
        pltpu.VMEM((CH, 8), jnp.float32),
        pltpu.VMEM_SHARED((NP, DQ), jnp.float32),
        pltpu.VMEM_SHARED((NC * NP, 8), jnp.float32),
        pltpu.SemaphoreType.DMA,
        pltpu.SemaphoreType.DMA,
        pltpu.SemaphoreType.DMA,
        pltpu.SemaphoreType.DMA,
    ],
    compiler_params=_sc_params,
)
def _sc_pass1(featA, featB, gidx, sidx, zq, z8, ones8,
              out_a, out_b, out_deg,
              gslab, sslab, buf0, buf1, bufb0, bufb1, onesb, acc, acc8,
              sem0, sem1, semb0, semb1):
    c = lax.axis_index("c")
    s = lax.axis_index("s")
    r0 = s * RPT
    pltpu.sync_copy(zq.at[pl.ds(r0, RPT)], acc.at[pl.ds(r0, RPT)])
    pltpu.sync_copy(z8.at[pl.ds(r0, RPT)], acc8.at[pl.ds(c * NP + r0, RPT)])
    pltpu.sync_copy(ones8, onesb)
    pltpu.sync_copy(gidx.at[c * NS + s], gslab)
    pltpu.sync_copy(sidx.at[s], sslab)
    plsc.subcore_barrier()

    pltpu.async_copy(featA.at[gslab.at[0]], buf0, sem0)

    @pl.when(c == 0)
    def _():
        pltpu.async_copy(featB.at[gslab.at[0]], bufb0, semb0)
        pltpu.async_copy(featB.at[gslab.at[1]], bufb1, semb1)

    def small(j, bb, sb):
        @pl.when(c == 0)
        def _():
            pltpu.make_async_copy(featB.at[gslab.at[j]], bb, sb).wait()
            pltpu.sync_copy(bb, acc8.at[sslab.at[j]], add=True)

            @pl.when(j + 2 < K2)
            def _():
                pltpu.async_copy(featB.at[gslab.at[j + 2]], bb, sb)

        @pl.when(c == 1)
        def _():
            pltpu.sync_copy(onesb, acc8.at[gslab.at[j]], add=True)

    def body(j2, carry):
        j = 2 * j2
        pltpu.async_copy(featA.at[gslab.at[j + 1]], buf1, sem1)
        pltpu.make_async_copy(featA.at[gslab.at[j]], buf0, sem0).wait()
        pltpu.sync_copy(buf0, acc.at[sslab.at[j]], add=True)
        small(j, bufb0, semb0)

        @pl.when(j + 2 < K2)
        def _():
            pltpu.async_copy(featA.at[gslab.at[j + 2]], buf0, sem0)

        pltpu.make_async_copy(featA.at[gslab.at[j + 1]], buf1, sem1).wait()
        pltpu.sync_copy(buf1, acc.at[sslab.at[j + 1]], add=True)
        small(j + 1, bufb1, semb1)
        return carry

    lax.fori_loop(0, K2 // 2, body, 0)
    plsc.subcore_barrier()
    pltpu.sync_copy(acc.at[pl.ds(r0, RPT)], out_a.at[pl.ds(c * NP + r0, RPT)])

    @pl.when(c == 0)
    def _():
        pltpu.sync_copy(acc8.at[pl.ds(r0, RPT)], out_b.at[pl.ds(r0, RPT)])

    @pl.when(c == 1)
    def _():
        pltpu.sync_copy(acc8.at[pl.ds(NP + r0, RPT)],
                        out_deg.at[pl.ds(r0, RPT)])


@functools.partial(
    pl.kernel,
    out_type=jax.ShapeDtypeStruct((NQ * NP, DQ), jnp.float32),
    mesh=_mesh,
    scratch_types=[
        pltpu.VMEM((K2, CH), jnp.int32),
        pltpu.VMEM((K2, CH), jnp.int32),
        pltpu.VMEM((CH, DQ), jnp.float32),
        pltpu.VMEM((CH, DQ), jnp.float32),
        pltpu.VMEM_SHARED((NP, DQ), jnp.float32),
        pltpu.SemaphoreType.DMA,
        pltpu.SemaphoreType.DMA,
    ],
    compiler_params=_sc_params,
)
def _sc_agg(table, gidx, sidx, zq, out, gslab, sslab, buf0, buf1, acc,
            sem0, sem1):
    c = lax.axis_index("c")
    s = lax.axis_index("s")
    r0 = s * RPT
    pltpu.sync_copy(sidx.at[s], sslab)
    for q in range(2):
        v = 2 * q + c
        pltpu.sync_copy(gidx.at[v * NS + s], gslab)
        pltpu.sync_copy(zq.at[pl.ds(r0, RPT)], acc.at[pl.ds(r0, RPT)])
        plsc.subcore_barrier()

        pltpu.async_copy(table.at[gslab.at[0]], buf0, sem0)

        def body(j2, carry):
            j = 2 * j2
            pltpu.async_copy(table.at[gslab.at[j + 1]], buf1, sem1)
            pltpu.make_async_copy(table.at[gslab.at[j]], buf0, sem0).wait()
            pltpu.sync_copy(buf0, acc.at[sslab.at[j]], add=True)

            @pl.when(j + 2 < K2)
            def _():
                pltpu.async_copy(table.at[gslab.at[j + 2]], buf0, sem0)

            pltpu.make_async_copy(table.at[gslab.at[j + 1]], buf1,
                                  sem1).wait()
            pltpu.sync_copy(buf1, acc.at[sslab.at[j + 1]], add=True)
            return carry

        lax.fori_loop(0, K2 // 2, body, 0)
        plsc.subcore_barrier()
        pltpu.sync_copy(acc.at[pl.ds(r0, RPT)],
                        out.at[pl.ds(v * NP + r0, RPT)])


def _split_q(ms_ref, ms):
    for j in range(NQ):
        ms_ref[j, :, :] = ms[:, j * DQ:(j + 1) * DQ]


def _cat_q(ref):
    return jnp.concatenate([ref[j] for j in range(NQ)], axis=1)


def _tc0_body(sa_ref, sb_ref, degs_ref, w_ref, ms_ref, rdeg_ref):
    sa = jnp.concatenate([sa_ref[0], sa_ref[1]], axis=1)
    sb = sb_ref[...]
    s2 = (jnp.sum(sa * sa, axis=1, keepdims=True)
          + jnp.sum(sb * sb, axis=1, keepdims=True))
    h = jnp.concatenate([sa, sb], axis=1) * lax.rsqrt(
        jnp.where(s2 > 0, s2, 1.0))
    deg = degs_ref[:, 0:1] + 1.0
    rdeg = lax.rsqrt(deg)
    m = jnp.dot(h, w_ref[...], preferred_element_type=jnp.float32)
    _split_q(ms_ref, m * rdeg)
    rdeg_ref[...] = jnp.broadcast_to(rdeg, (BR, 128))


def _tc_layer0(sumsa, sumsb, degs, w0p):
    return pl.pallas_call(
        _tc0_body,
        grid=(GRID,),
        in_specs=[
            pl.BlockSpec((NC, BR, DQ), lambda i: (0, i, 0)),
            pl.BlockSpec((BR, 8), lambda i: (i, 0)),
            pl.BlockSpec((BR, 8), lambda i: (i, 0)),
            pl.BlockSpec((D1, D2), lambda i: (0, 0)),
        ],
        out_specs=[
            pl.BlockSpec((NQ, BR, DQ), lambda i: (0, i, 0)),
            pl.BlockSpec((BR, 128), lambda i: (i, 0)),
        ],
        out_shape=[
            jax.ShapeDtypeStruct((NQ, NP, DQ), jnp.float32),
            jax.ShapeDtypeStruct((NP, 128), jnp.float32),
        ],
    )(sumsa, sumsb, degs, w0p)


def _tck_body(agg_ref, msp_ref, rdeg_ref, w_ref, b_ref, ms_ref):
    rdeg = rdeg_ref[:, 0:1]
    pre = (_cat_q(agg_ref) + _cat_q(msp_ref)) * rdeg + b_ref[0:1, :]
    h = jnp.maximum(pre, 0.0)
    m = jnp.dot(h, w_ref[...], preferred_element_type=jnp.float32)
    _split_q(ms_ref, m * rdeg)


def _tc_layer(agg, msp, rdegb, w, b):
    return pl.pallas_call(
        _tck_body,
        grid=(GRID,),
        in_specs=[
            pl.BlockSpec((NQ, BR, DQ), lambda i: (0, i, 0)),
            pl.BlockSpec((NQ, BR, DQ), lambda i: (0, i, 0)),
            pl.BlockSpec((BR, 128), lambda i: (i, 0)),
            pl.BlockSpec((D2, D2), lambda i: (0, 0)),
            pl.BlockSpec((1, D2), lambda i: (0, 0)),
        ],
        out_specs=pl.BlockSpec((NQ, BR, DQ), lambda i: (0, i, 0)),
        out_shape=jax.ShapeDtypeStruct((NQ, NP, DQ), jnp.float32),
    )(agg, msp, rdegb, w, b)


def _tcf_body(agg_ref, msp_ref, rdeg_ref, b_ref, wl_ref, bl_ref, batch_ref,
              out_ref, pacc, cacc):
    i = pl.program_id(0)
    rdeg = rdeg_ref[:, 0:1]
    pre = (_cat_q(agg_ref) + _cat_q(msp_ref)) * rdeg + b_ref[0:1, :]
    h = jnp.maximum(pre, 0.0)
    bt = batch_ref[0, 0, :]
    gid = lax.broadcasted_iota(jnp.int32, (BR, G), 1)
    oh = (bt[:, None] == gid).astype(jnp.float32)
    pc = lax.dot_general(oh, h, (((0,), (0,)), ((), ())),
                         preferred_element_type=jnp.float32,
                         precision=lax.Precision.HIGHEST)
    ones = jnp.ones((BR, 128), jnp.float32)
    cn = lax.dot_general(oh, ones, (((0,), (0,)), ((), ())),
                         preferred_element_type=jnp.float32,
                         precision=lax.Precision.HIGHEST)

    @pl.when(i == 0)
    def _():
        pacc[...] = jnp.zeros((G, D2), jnp.float32)
        cacc[...] = jnp.zeros((G, 128), jnp.float32)

    pacc[...] += pc
    cacc[...] += cn

    @pl.when(i == GRID - 1)
    def _():
        pooled = pacc[...] / jnp.maximum(cacc[:, 0:1], 1.0)
        out_ref[...] = jnp.dot(
            pooled, wl_ref[...],
            preferred_element_type=jnp.float32) + bl_ref[...]


def _tc_final(agg, msp, rdegb, b, wlp, blb, batchp):
    return pl.pallas_call(
        _tcf_body,
        grid=(GRID,),
        in_specs=[
            pl.BlockSpec((NQ, BR, DQ), lambda i: (0, i, 0)),
            pl.BlockSpec((NQ, BR, DQ), lambda i: (0, i, 0)),
            pl.BlockSpec((BR, 128), lambda i: (i, 0)),
            pl.BlockSpec((1, D2), lambda i: (0, 0)),
            pl.BlockSpec((D2, 128), lambda i: (0, 0)),
            pl.BlockSpec((1, 128), lambda i: (0, 0)),
            pl.BlockSpec((1, 1, BR), lambda i: (i, 0, 0)),
        ],
        out_specs=pl.BlockSpec((G, 128), lambda i: (0, 0)),
        out_shape=jax.ShapeDtypeStruct((G, 128), jnp.float32),
        scratch_shapes=[
            pltpu.VMEM((G, D2), jnp.float32),
            pltpu.VMEM((G, 128), jnp.float32),
        ],
    )(agg, msp, rdegb, b, wlp, blb, batchp)


def kernel(x, pos, z, W0, b0, W1, b1, W2, b2, Wl, bl, edge_index, batch):
    f32 = jnp.float32
    xh = jnp.pad(x, ((0, NP - N), (0, 0))).reshape(
        NP, NC, DQ).transpose(1, 0, 2).reshape(NC * NP, DQ)
    featsB = jnp.pad(jnp.concatenate([pos, z[:, None]], axis=1),
                     ((0, NP - N), (0, 4)))
    src = jnp.pad(edge_index[0], (0, EP - E), constant_values=N)
    dst = jnp.pad(edge_index[1], (0, EP - E), constant_values=N)
    gidx1 = (dst[None, :]
             + (jnp.arange(NC, dtype=jnp.int32) * NP)[:, None]).reshape(
                 NC * NS, K2, CH)
    sidx1 = src.reshape(NS, K2, CH)
    gidx2 = (src[None, :]
             + (jnp.arange(NQ, dtype=jnp.int32) * NP)[:, None]).reshape(
                 NQ * NS, K2, CH)
    sidx2 = dst.reshape(NS, K2, CH)
    z8 = jnp.zeros((NP, 8), f32)
    zq = jnp.zeros((NP, DQ), f32)
    ones8 = jnp.zeros((CH, 8), f32).at[:, 0].set(1.0)
    W0p = jnp.pad(W0, ((0, D1 - 132), (0, 0)))
    Wlp = jnp.pad(Wl, ((0, 0), (0, 127)))
    blb = jnp.broadcast_to(bl.reshape(1, 1), (1, 128))
    batchP = jnp.pad(batch, (0, NP - N), constant_values=127).reshape(
        GRID, 1, BR)

    out_a, out_b, out_deg = _sc_pass1(xh, featsB, gidx1, sidx1,
                                      zq, z8, ones8)
    ms, rdegb = _tc_layer0(out_a.reshape(NC, NP, DQ), out_b, out_deg, W0p)

    for W_, b_ in ((W1, b0), (W2, b1)):
        agg = _sc_agg(ms.reshape(NQ * NP, DQ), gidx2, sidx2, zq)
        ms = _tc_layer(agg.reshape(NQ, NP, DQ), ms, rdegb, W_,
                       b_.reshape(1, D2))

    agg = _sc_agg(ms.reshape(NQ * NP, DQ), gidx2, sidx2, zq)
    res = _tc_final(agg.reshape(NQ, NP, DQ), ms, rdegb, b2.reshape(1, D2),
                    Wlp, blb, batchP)
    return res[:, 0]
